# baseline (speedup 1.0000x reference)
import jax
import jax.numpy as jnp
from jax import lax
from jax.experimental import pallas as pl
from jax.experimental.pallas import tpu as pltpu
from jax.experimental.pallas import tpu_sc as plsc

_D = 64
_BATCH = 16384
_SEQ = 50
_L = 16
_NC, _NS = 2, 16
_NW = _NC * _NS
_NB = _BATCH // _NW
_CH = 128
_SUBS = _NB // _CH
_CPW = _SEQ * _SUBS
_SCALE = 8.0


def _gather_body(tbl_hbm, xT_hbm, out_hbm, xb_v, li_v, g_v, o_v,
                 g_sem, w_sem):
    wid = lax.axis_index("s") * _NC + lax.axis_index("c")
    b0 = wid * _NB
    pltpu.sync_copy(xT_hbm.at[:, pl.ds(b0, _NB)], xb_v)

    def prep_and_fire(c, bf):
        s = c // _SUBS
        sub = c % _SUBS
        for k in range(_CH // _L):
            v = xb_v[s, pl.ds(sub * _CH + k * _L, _L)]
            li_v[bf, pl.ds(k * _L, _L)] = lax.shift_right_logical(v, 1)
        pltpu.async_copy(tbl_hbm.at[li_v.at[bf]], g_v.at[bf], g_sem.at[bf])

    def wait_gather(bf):
        pltpu.make_async_copy(
            tbl_hbm.at[li_v.at[bf]], g_v.at[bf], g_sem.at[bf]).wait()

    def fire_write(c, bf):
        s = c // _SUBS
        sub = c % _SUBS
        pltpu.async_copy(
            o_v.at[bf],
            out_hbm.at[s, :, pl.ds(b0 + sub * _CH, _CH)],
            w_sem.at[bf])

    def wait_write(bf):
        pltpu.make_async_copy(
            o_v.at[bf], out_hbm.at[0, :, pl.ds(b0, _CH)], w_sem.at[bf]).wait()

    def transpose_scale(c, bf):
        s = c // _SUBS
        sub = c % _SUBS
        for g in range(_CH // _L):
            hvec = (xb_v[s, pl.ds(sub * _CH + g * _L, _L)] & 1) * _D
            lam = lax.iota(jnp.int32, _L) + (g * _L)

            def e_body(e4, carry, _hvec=hvec, _lam=lam, _g=g, _bf=bf):
                for u in range(4):
                    e = e4 * 4 + u
                    col = _hvec + e
                    vec = plsc.load_gather(g_v.at[_bf], [_lam, col])
                    o_v[_bf, e, pl.ds(_g * _L, _L)] = vec * _SCALE
                return carry

            lax.fori_loop(0, _D // 4, e_body, 0)

    prep_and_fire(0, 0)

    def pair(p, carry):
        for b in range(2):
            c = p * 2 + b

            @pl.when(c + 1 < _CPW)
            def _(c=c, nb=1 - b):
                prep_and_fire(c + 1, nb)

            wait_gather(b)

            @pl.when(c >= 2)
            def _(b=b):
                wait_write(b)

            transpose_scale(c, b)
            fire_write(c, b)
        return carry

    lax.fori_loop(0, _CPW // 2, pair, 0)
    wait_write(0)
    wait_write(1)


def kernel(x, lut):
    xT = jnp.transpose(x).astype(jnp.int32)
    tbl = lut.reshape(500000, 128)
    mesh = plsc.VectorSubcoreMesh(core_axis_name="c", subcore_axis_name="s")
    sc_call = pl.kernel(
        _gather_body,
        mesh=mesh,
        out_type=jax.ShapeDtypeStruct((_SEQ, _D, _BATCH), jnp.float32),
        scratch_types=[
            pltpu.VMEM((_SEQ, _NB), jnp.int32),
            pltpu.VMEM((2, _CH), jnp.int32),
            pltpu.VMEM((2, _CH, 128), jnp.float32),
            pltpu.VMEM((2, _D, _CH), jnp.float32),
            pltpu.SemaphoreType.DMA((2,)),
            pltpu.SemaphoreType.DMA((2,)),
        ],
        compiler_params=pltpu.CompilerParams(
            use_tc_tiling_on_sc=True, needs_layout_passes=False),
    )
    out = sc_call(tbl, xT)
    return out.transpose(2, 0, 1)


# speedup vs baseline: 1.5277x; 1.5277x over previous
import jax
import jax.numpy as jnp
from jax import lax
from jax.experimental import pallas as pl
from jax.experimental.pallas import tpu as pltpu
from jax.experimental.pallas import tpu_sc as plsc

_D = 64
_BATCH = 16384
_SEQ = 50
_L = 16
_NC, _NS = 2, 16
_NW = _NC * _NS
_NB = _BATCH // _NW
_CH = 128
_SUBS = _NB // _CH
_CPW = _SEQ * _SUBS
_SCALE = 8.0


def _gather_body(tbl_hbm, xT_hbm, out_hbm, xb_v, li_v, g_v, o_v,
                 g_sem, w_sem):
    wid = lax.axis_index("s") * _NC + lax.axis_index("c")
    b0 = wid * _NB
    pltpu.sync_copy(xT_hbm.at[:, pl.ds(b0, _NB)], xb_v)

    def prep_and_fire(c, bf):
        s = c // _SUBS
        sub = c % _SUBS
        for k in range(_CH // _L):
            v = xb_v[s, pl.ds(sub * _CH + k * _L, _L)]
            li_v[bf, pl.ds(k * _L, _L)] = lax.shift_right_logical(v, 1)
        pltpu.async_copy(tbl_hbm.at[li_v.at[bf]], g_v.at[bf], g_sem.at[bf])

    def wait_gather(bf):
        pltpu.make_async_copy(
            tbl_hbm.at[li_v.at[bf]], g_v.at[bf], g_sem.at[bf]).wait()

    def fire_write(c, bf):
        s = c // _SUBS
        sub = c % _SUBS
        pltpu.async_copy(
            o_v.at[bf],
            out_hbm.at[s, :, pl.ds(b0 + sub * _CH, _CH)],
            w_sem.at[bf])

    def wait_write(bf):
        pltpu.make_async_copy(
            o_v.at[bf], out_hbm.at[0, :, pl.ds(b0, _CH)], w_sem.at[bf]).wait()

    def transpose_scale(c, bf):
        s = c // _SUBS
        sub = c % _SUBS
        for g in range(_CH // _L):
            hvec = (xb_v[s, pl.ds(sub * _CH + g * _L, _L)] & 1) * _D
            lam = lax.iota(jnp.int32, _L) + (g * _L)

            @plsc.parallel_loop(0, _D, unroll=8)
            def e_body(e, _hvec=hvec, _lam=lam, _g=g, _bf=bf):
                col = _hvec + e
                vec = plsc.load_gather(g_v.at[_bf], [_lam, col])
                o_v[_bf, e, pl.ds(_g * _L, _L)] = vec * _SCALE

    prep_and_fire(0, 0)

    def pair(p, carry):
        for b in range(2):
            c = p * 2 + b

            @pl.when(c + 1 < _CPW)
            def _(c=c, nb=1 - b):
                prep_and_fire(c + 1, nb)

            wait_gather(b)

            @pl.when(c >= 2)
            def _(b=b):
                wait_write(b)

            transpose_scale(c, b)
            fire_write(c, b)
        return carry

    lax.fori_loop(0, _CPW // 2, pair, 0)
    wait_write(0)
    wait_write(1)


def kernel(x, lut):
    xT = jnp.transpose(x).astype(jnp.int32)
    tbl = lut.reshape(500000, 128)
    mesh = plsc.VectorSubcoreMesh(core_axis_name="c", subcore_axis_name="s")
    sc_call = pl.kernel(
        _gather_body,
        mesh=mesh,
        out_type=jax.ShapeDtypeStruct((_SEQ, _D, _BATCH), jnp.float32),
        scratch_types=[
            pltpu.VMEM((_SEQ, _NB), jnp.int32),
            pltpu.VMEM((2, _CH), jnp.int32),
            pltpu.VMEM((2, _CH, 128), jnp.float32),
            pltpu.VMEM((2, _D, _CH), jnp.float32),
            pltpu.SemaphoreType.DMA((2,)),
            pltpu.SemaphoreType.DMA((2,)),
        ],
        compiler_params=pltpu.CompilerParams(
            use_tc_tiling_on_sc=True, needs_layout_passes=False),
    )
    out = sc_call(tbl, xT)
    return out.transpose(2, 0, 1)


# speedup vs baseline: 2.0881x; 1.3668x over previous
import jax
import jax.numpy as jnp
from jax import lax
from jax.experimental import pallas as pl
from jax.experimental.pallas import tpu as pltpu
from jax.experimental.pallas import tpu_sc as plsc

_D = 64
_BATCH = 16384
_SEQ = 50
_L = 16
_NC, _NS = 2, 16
_NW = _NC * _NS
_NB = _BATCH // _NW
_CH = 128
_SUBS = _NB // _CH
_CPW = _SEQ * _SUBS
_SCALE = 8.0


def _gather_body(tbl_hbm, xT_hbm, out_hbm, xb_v, li_v, g_v, o_v,
                 g_sem, w_sem):
    wid = lax.axis_index("s") * _NC + lax.axis_index("c")
    b0 = wid * _NB
    pltpu.sync_copy(xT_hbm.at[:, pl.ds(b0, _NB)], xb_v)

    def prep_and_fire(c, bf):
        s = c // _SUBS
        sub = c % _SUBS
        for k in range(_CH // _L):
            v = xb_v[s, pl.ds(sub * _CH + k * _L, _L)]
            li_v[bf, pl.ds(k * _L, _L)] = lax.shift_right_logical(v, 1)
        pltpu.async_copy(tbl_hbm.at[li_v.at[bf]], g_v.at[bf], g_sem.at[bf])

    def wait_gather(bf):
        pltpu.make_async_copy(
            tbl_hbm.at[li_v.at[bf]], g_v.at[bf], g_sem.at[bf]).wait()

    def fire_write(c, bf):
        s = c // _SUBS
        sub = c % _SUBS
        pltpu.async_copy(
            o_v.at[bf],
            out_hbm.at[s, :, pl.ds(b0 + sub * _CH, _CH)],
            w_sem.at[bf])

    def wait_write(bf):
        pltpu.make_async_copy(
            o_v.at[bf], out_hbm.at[0, :, pl.ds(b0, _CH)], w_sem.at[bf]).wait()

    def transpose_scale(c, bf):
        s = c // _SUBS
        sub = c % _SUBS
        for g in range(_CH // _L):
            hvec = (xb_v[s, pl.ds(sub * _CH + g * _L, _L)] & 1) * _D
            lam = lax.iota(jnp.int32, _L) + (g * _L)
            for k in range(_D // _L):
                hk = hvec + (k * _L)

                @plsc.parallel_loop(0, _L, unroll=8)
                def d_body(d, _hk=hk, _lam=lam, _k=k, _bf=bf):
                    j = (lax.iota(jnp.int32, _L) + d) & (_L - 1)
                    vec = plsc.load_gather(g_v.at[_bf], [_lam, _hk + j])
                    plsc.store_scatter(
                        o_v.at[_bf], [j + (_k * _L), _lam], vec * _SCALE)

    prep_and_fire(0, 0)

    def pair(p, carry):
        for b in range(2):
            c = p * 2 + b

            @pl.when(c + 1 < _CPW)
            def _(c=c, nb=1 - b):
                prep_and_fire(c + 1, nb)

            wait_gather(b)

            @pl.when(c >= 2)
            def _(b=b):
                wait_write(b)

            transpose_scale(c, b)
            fire_write(c, b)
        return carry

    lax.fori_loop(0, _CPW // 2, pair, 0)
    wait_write(0)
    wait_write(1)


def kernel(x, lut):
    xT = jnp.transpose(x).astype(jnp.int32)
    tbl = lut.reshape(500000, 128)
    mesh = plsc.VectorSubcoreMesh(core_axis_name="c", subcore_axis_name="s")
    sc_call = pl.kernel(
        _gather_body,
        mesh=mesh,
        out_type=jax.ShapeDtypeStruct((_SEQ, _D, _BATCH), jnp.float32),
        scratch_types=[
            pltpu.VMEM((_SEQ, _NB), jnp.int32),
            pltpu.VMEM((2, _CH), jnp.int32),
            pltpu.VMEM((2, _CH, 128), jnp.float32),
            pltpu.VMEM((2, _D, _CH), jnp.float32),
            pltpu.SemaphoreType.DMA((2,)),
            pltpu.SemaphoreType.DMA((2,)),
        ],
        compiler_params=pltpu.CompilerParams(
            use_tc_tiling_on_sc=True, needs_layout_passes=False),
    )
    out = sc_call(tbl, xT)
    return out.transpose(2, 0, 1)


# speedup vs baseline: 2.8092x; 1.3453x over previous
import jax
import jax.numpy as jnp
from jax import lax
from jax.experimental import pallas as pl
from jax.experimental.pallas import tpu as pltpu
from jax.experimental.pallas import tpu_sc as plsc

_D = 64
_BATCH = 16384
_SEQ = 50
_L = 16
_NC, _NS = 2, 16
_NW = _NC * _NS
_NB = _BATCH // _NW
_CH = 128
_SUBS = _NB // _CH
_CPW = _SEQ * _SUBS
_SCALE = 8.0


_APW = 244


def _transpose_pairs(src_ref, o_v, dst_bf, nlines):
    for pg in range(nlines // _L):
        for cg in range(8):
            erow0 = _L * (cg % 4)
            base_col = 32 * pg + (cg // 4)

            @plsc.parallel_loop(0, _L, unroll=8)
            def d_body(d, _erow0=erow0, _bc=base_col, _pg=pg, _cg=cg,
                       _bf=dst_bf):
                i = lax.iota(jnp.int32, _L)
                j = (i + d) & (_L - 1)
                vec = plsc.load_gather(src_ref, [_erow0 + j, _bc + 2 * i])
                plsc.store_scatter(
                    o_v.at[_bf], [_L * _pg + i, _L * _cg + j], vec)


def _conv_body(lutT_hbm, tbl_hbm, s_v, o_v, st_v, r_sem, w_sem):
    wid = lax.axis_index("s") * _NC + lax.axis_index("c")
    base = wid * _APW

    def fire_read(blk, bf):
        pltpu.async_copy(
            lutT_hbm.at[:, pl.ds(blk * 128, 128)], s_v.at[bf], r_sem.at[bf])

    def wait_read(bf):
        pltpu.make_async_copy(
            lutT_hbm.at[:, pl.ds(0, 128)], s_v.at[bf], r_sem.at[bf]).wait()

    def fire_write(blk, bf):
        pltpu.async_copy(
            o_v.at[bf], tbl_hbm.at[pl.ds(blk * 64, 64)], w_sem.at[bf])

    def wait_write(bf):
        pltpu.make_async_copy(
            o_v.at[bf], tbl_hbm.at[pl.ds(0, 64)], w_sem.at[bf]).wait()

    fire_read(base, 0)

    def pair(pp, carry):
        for b in range(2):
            i = pp * 2 + b
            blk = base + i

            @pl.when(i + 1 < _APW)
            def _(blk=blk, nb=1 - b):
                fire_read(blk + 1, nb)

            wait_read(b)

            @pl.when(i >= 2)
            def _(b=b):
                wait_write(b)

            _transpose_pairs(s_v.at[b], o_v, b, 64)
            fire_write(blk, b)
        return carry

    lax.fori_loop(0, _APW // 2, pair, 0)
    wait_write(0)
    wait_write(1)

    @pl.when(wid < 4)
    def _():
        blk = 32 * _APW + wid
        fire_read(blk, 0)
        wait_read(0)
        _transpose_pairs(s_v.at[0], o_v, 0, 64)
        fire_write(blk, 0)
        wait_write(0)

    @pl.when(wid == 4)
    def _():
        pltpu.async_copy(
            lutT_hbm.at[:, pl.ds(999936, 64)], st_v, r_sem.at[0])
        pltpu.make_async_copy(
            lutT_hbm.at[:, pl.ds(999936, 64)], st_v, r_sem.at[0]).wait()
        _transpose_pairs(st_v, o_v, 0, 32)
        pltpu.async_copy(
            o_v.at[0, pl.ds(0, 32)], tbl_hbm.at[pl.ds(499968, 32)],
            w_sem.at[0])
        pltpu.make_async_copy(
            o_v.at[0, pl.ds(0, 32)], tbl_hbm.at[pl.ds(499968, 32)],
            w_sem.at[0]).wait()


def _gather_body(tbl_hbm, xT_hbm, out_hbm, xb_v, li_v, g_v, o_v,
                 g_sem, w_sem):
    wid = lax.axis_index("s") * _NC + lax.axis_index("c")
    b0 = wid * _NB
    pltpu.sync_copy(xT_hbm.at[:, pl.ds(b0, _NB)], xb_v)

    def prep_and_fire(c, bf):
        s = c // _SUBS
        sub = c % _SUBS
        for k in range(_CH // _L):
            v = xb_v[s, pl.ds(sub * _CH + k * _L, _L)]
            li_v[bf, pl.ds(k * _L, _L)] = lax.shift_right_logical(v, 1)
        pltpu.async_copy(tbl_hbm.at[li_v.at[bf]], g_v.at[bf], g_sem.at[bf])

    def wait_gather(bf):
        pltpu.make_async_copy(
            tbl_hbm.at[li_v.at[bf]], g_v.at[bf], g_sem.at[bf]).wait()

    def fire_write(c, bf):
        s = c // _SUBS
        sub = c % _SUBS
        pltpu.async_copy(
            o_v.at[bf],
            out_hbm.at[s, :, pl.ds(b0 + sub * _CH, _CH)],
            w_sem.at[bf])

    def wait_write(bf):
        pltpu.make_async_copy(
            o_v.at[bf], out_hbm.at[0, :, pl.ds(b0, _CH)], w_sem.at[bf]).wait()

    def transpose_scale(c, bf):
        s = c // _SUBS
        sub = c % _SUBS
        for g in range(_CH // _L):
            hvec = (xb_v[s, pl.ds(sub * _CH + g * _L, _L)] & 1) * _D
            lam = lax.iota(jnp.int32, _L) + (g * _L)
            for k in range(_D // _L):
                hk = hvec + (k * _L)

                @plsc.parallel_loop(0, _L, unroll=8)
                def d_body(d, _hk=hk, _lam=lam, _k=k, _bf=bf):
                    j = (lax.iota(jnp.int32, _L) + d) & (_L - 1)
                    vec = plsc.load_gather(g_v.at[_bf], [_lam, _hk + j])
                    plsc.store_scatter(
                        o_v.at[_bf], [j + (_k * _L), _lam], vec * _SCALE)

    prep_and_fire(0, 0)

    def pair(p, carry):
        for b in range(2):
            c = p * 2 + b

            @pl.when(c + 1 < _CPW)
            def _(c=c, nb=1 - b):
                prep_and_fire(c + 1, nb)

            wait_gather(b)

            @pl.when(c >= 2)
            def _(b=b):
                wait_write(b)

            transpose_scale(c, b)
            fire_write(c, b)
        return carry

    lax.fori_loop(0, _CPW // 2, pair, 0)
    wait_write(0)
    wait_write(1)


def kernel(x, lut):
    xT = jnp.transpose(x).astype(jnp.int32)
    mesh = plsc.VectorSubcoreMesh(core_axis_name="c", subcore_axis_name="s")
    conv_call = pl.kernel(
        _conv_body,
        mesh=mesh,
        out_type=jax.ShapeDtypeStruct((500000, 128), jnp.float32),
        scratch_types=[
            pltpu.VMEM((2, _D, 128), jnp.float32),
            pltpu.VMEM((2, _D, 128), jnp.float32),
            pltpu.VMEM((_D, _D), jnp.float32),
            pltpu.SemaphoreType.DMA((2,)),
            pltpu.SemaphoreType.DMA((2,)),
        ],
        compiler_params=pltpu.CompilerParams(
            use_tc_tiling_on_sc=True, needs_layout_passes=False),
    )
    tbl = conv_call(jnp.transpose(lut))
    sc_call = pl.kernel(
        _gather_body,
        mesh=mesh,
        out_type=jax.ShapeDtypeStruct((_SEQ, _D, _BATCH), jnp.float32),
        scratch_types=[
            pltpu.VMEM((_SEQ, _NB), jnp.int32),
            pltpu.VMEM((2, _CH), jnp.int32),
            pltpu.VMEM((2, _CH, 128), jnp.float32),
            pltpu.VMEM((2, _D, _CH), jnp.float32),
            pltpu.SemaphoreType.DMA((2,)),
            pltpu.SemaphoreType.DMA((2,)),
        ],
        compiler_params=pltpu.CompilerParams(
            use_tc_tiling_on_sc=True, needs_layout_passes=False),
    )
    out = sc_call(tbl, xT)
    return out.transpose(2, 0, 1)
